# baseline (speedup 1.0000x reference)
import functools

import jax
import jax.numpy as jnp
from jax import lax
from jax.experimental import pallas as pl
from jax.experimental.pallas import tpu as pltpu
from jax.experimental.pallas import tpu_sc as plsc

N = 10000
D = 128
E = 320000
HID = 128

NC = 2
NS = 16
L = 16

EPT = E // NS
CH = 800
NCH = EPT // CH
RCHUNK = 32
NSUB = CH // RCHUNK
NPAD = 10240
ROWS_T = NPAD // NS
ZROWS = 128


def _sc_edge_body(el_hbm, er_hbm, src_hbm, dst_hbm, feat_hbm,
                  out_hbm,
                  el_v, er_v, s_v, isrc_c, idst_c, a_c, csrc0, cdst0,
                  rows0, shared_s, shared_out, gsem0):
    cid = lax.axis_index("c")
    sid = lax.axis_index("s")

    for lp in range(2):
        p = cid * 2 + lp
        ebase = p * E + sid * EPT

        pltpu.sync_copy(el_hbm.at[pl.ds(p * N, N)], el_v)
        pltpu.sync_copy(er_hbm.at[pl.ds(p * N, N)], er_v)

        def _zs(i, c):
            s_v[pl.ds(i * 16, 16)] = jnp.zeros((16,), jnp.float32)
            return c
        lax.fori_loop(0, N // 16, _zs, 0)

        def _zr(i, c):
            for j in range(8):
                rows0[i, pl.ds(j * 16, 16)] = jnp.zeros((16,), jnp.float32)
            return c
        lax.fori_loop(0, RCHUNK, _zr, 0)

        @pl.when(sid == 0)
        def _():
            pltpu.sync_copy(s_v, shared_s)
        for k in range(ROWS_T // RCHUNK):
            pltpu.sync_copy(
                rows0,
                shared_out.at[pl.ds(sid * ROWS_T + k * RCHUNK, RCHUNK)])
        plsc.subcore_barrier()

        def _sch(ci, c):
            off = ebase + ci * CH
            pltpu.sync_copy(src_hbm.at[pl.ds(off, CH)], isrc_c)
            pltpu.sync_copy(dst_hbm.at[pl.ds(off, CH)], idst_c)

            def _sg(k, c2):
                s16 = isrc_c[pl.ds(k * 16, 16)]
                d16 = idst_c[pl.ds(k * 16, 16)]
                ev = (plsc.load_gather(el_v, [s16])
                      + plsc.load_gather(er_v, [d16]))
                ev = jnp.where(ev >= 0.0, ev, 0.01 * ev)
                a_c[pl.ds(k * 16, 16)] = jnp.exp(ev)
                return c2
            lax.fori_loop(0, CH // 16, _sg, 0)

            def _ssc(si, c2):
                soff = si * RCHUNK
                for k in range(RCHUNK // 16):
                    cdst0[pl.ds(k * 16, 16)] = idst_c[pl.ds(soff + k * 16, 16)]
                pltpu.sync_copy(a_c.at[pl.ds(soff, RCHUNK)],
                                shared_s.at[cdst0], add=True)
                return c2
            lax.fori_loop(0, CH // RCHUNK, _ssc, 0)
            return c
        lax.fori_loop(0, NCH, _sch, 0)

        plsc.subcore_barrier()
        pltpu.sync_copy(shared_s, s_v)

        fbase = cid * N

        def _rch(ci, c):
            off = ebase + ci * CH
            pltpu.sync_copy(src_hbm.at[pl.ds(off, CH)], isrc_c)
            pltpu.sync_copy(dst_hbm.at[pl.ds(off, CH)], idst_c)

            def _ag(k, c2):
                s16 = isrc_c[pl.ds(k * 16, 16)]
                d16 = idst_c[pl.ds(k * 16, 16)]
                ev = (plsc.load_gather(el_v, [s16])
                      + plsc.load_gather(er_v, [d16]))
                ev = jnp.where(ev >= 0.0, ev, 0.01 * ev)
                sg = plsc.load_gather(s_v, [d16])
                a_c[pl.ds(k * 16, 16)] = jnp.exp(ev) / (sg + 1e-9)
                return c2
            lax.fori_loop(0, CH // 16, _ag, 0)

            def _sub(si, c2):
                soff = si * RCHUNK
                for k in range(RCHUNK // 16):
                    sl = pl.ds(k * 16, 16)
                    csrc0[sl] = isrc_c[pl.ds(soff + k * 16, 16)] + fbase
                    cdst0[sl] = idst_c[pl.ds(soff + k * 16, 16)]
                pltpu.async_copy(feat_hbm.at[csrc0], rows0, gsem0).wait()

                def _sc(i, c3):
                    asp = plsc.load_gather(
                        a_c, [jnp.full((16,), soff + i, jnp.int32)])
                    for j in range(8):
                        sl = pl.ds(j * 16, 16)
                        rows0[i, sl] = rows0[i, sl] * asp
                    return c3
                lax.fori_loop(0, RCHUNK, _sc, 0)
                pltpu.sync_copy(rows0, shared_out.at[cdst0], add=True)
                return c2
            lax.fori_loop(0, CH // RCHUNK, _sub, 0)
            return c
        lax.fori_loop(0, NCH, _rch, 0)

        plsc.subcore_barrier()
        for k in range(ROWS_T // ZROWS):
            r0 = sid * ROWS_T + k * ZROWS
            pltpu.sync_copy(shared_out.at[pl.ds(r0, ZROWS)],
                            out_hbm.at[p, pl.ds(r0, ZROWS)])
        plsc.subcore_barrier()


_sc_edge = functools.partial(
    pl.kernel,
    out_type=jax.ShapeDtypeStruct((4, NPAD, D), jnp.float32),
    mesh=plsc.VectorSubcoreMesh(core_axis_name="c", subcore_axis_name="s"),
    compiler_params=pltpu.CompilerParams(needs_layout_passes=False),
    scratch_types=[
        pltpu.VMEM((N,), jnp.float32),
        pltpu.VMEM((N,), jnp.float32),
        pltpu.VMEM((N,), jnp.float32),
        pltpu.VMEM((CH,), jnp.int32),
        pltpu.VMEM((CH,), jnp.int32),
        pltpu.VMEM((CH,), jnp.float32),
        pltpu.VMEM((RCHUNK,), jnp.int32),
        pltpu.VMEM((RCHUNK,), jnp.int32),
        pltpu.VMEM((RCHUNK, D), jnp.float32),
        pltpu.VMEM_SHARED((N,), jnp.float32),
        pltpu.VMEM_SHARED((NPAD, D), jnp.float32),
        pltpu.SemaphoreType.DMA,
    ],
)(_sc_edge_body)


BN_A = 2000


def _proj_body(fu_ref, fi_ref, wu_ref, wi_ref, pu_ref, pi_ref):
    pu_ref[...] = jnp.dot(fu_ref[...], wu_ref[...],
                          preferred_element_type=jnp.float32)
    pi_ref[...] = jnp.dot(fi_ref[...], wi_ref[...],
                          preferred_element_type=jnp.float32)


_proj = pl.pallas_call(
    _proj_body,
    grid=(N // BN_A,),
    in_specs=[pl.BlockSpec((BN_A, D), lambda i: (i, 0)),
              pl.BlockSpec((BN_A, D), lambda i: (i, 0)),
              pl.BlockSpec((D, 4), lambda i: (0, 0)),
              pl.BlockSpec((D, 4), lambda i: (0, 0))],
    out_specs=[pl.BlockSpec((BN_A, 4), lambda i: (i, 0)),
               pl.BlockSpec((BN_A, 4), lambda i: (i, 0))],
    out_shape=[jax.ShapeDtypeStruct((N, 4), jnp.float32),
               jax.ShapeDtypeStruct((N, 4), jnp.float32)],
)

BN_B = 1000


def _comb_body(part_ref, w1_ref, b1_ref, w2_ref, h_ref, w_ref):
    i = pl.program_id(0)

    @pl.when(i == 0)
    def _():
        w_ref[...] = jnp.zeros_like(w_ref)

    for p in range(4):
        x = part_ref[p]
        h = jnp.where(x > 0.0, x, jnp.exp(x) - 1.0)
        h_ref[p] = h
        t = jnp.tanh(jnp.dot(h, w1_ref[...],
                             preferred_element_type=jnp.float32)
                     + b1_ref[...])
        contrib = jnp.sum(t * w2_ref[...])
        w_ref[pl.ds(p, 1), :] = w_ref[pl.ds(p, 1), :] + contrib


_comb = pl.pallas_call(
    _comb_body,
    grid=(N // BN_B,),
    in_specs=[pl.BlockSpec((4, BN_B, D), lambda i: (0, i, 0)),
              pl.BlockSpec((D, HID), lambda i: (0, 0)),
              pl.BlockSpec((1, HID), lambda i: (0, 0)),
              pl.BlockSpec((1, HID), lambda i: (0, 0))],
    out_specs=[pl.BlockSpec((4, BN_B, D), lambda i: (0, i, 0)),
               pl.BlockSpec((4, 128), lambda i: (0, 0))],
    out_shape=[jax.ShapeDtypeStruct((4, N, D), jnp.float32),
               jax.ShapeDtypeStruct((4, 128), jnp.float32)],
)

BN_F = 2000


def _fin_body(h_ref, w_ref, u_ref, i_ref):
    wv = w_ref[...] * (1.0 / N)
    wu0, wu1 = wv[0:1], wv[1:2]
    mu = jnp.maximum(wu0, wu1)
    e0, e1 = jnp.exp(wu0 - mu), jnp.exp(wu1 - mu)
    u_ref[...] = h_ref[0] * (e0 / (e0 + e1)) + h_ref[1] * (e1 / (e0 + e1))
    wi0, wi1 = wv[2:3], wv[3:4]
    mi = jnp.maximum(wi0, wi1)
    f0, f1 = jnp.exp(wi0 - mi), jnp.exp(wi1 - mi)
    i_ref[...] = h_ref[2] * (f0 / (f0 + f1)) + h_ref[3] * (f1 / (f0 + f1))


_fin = pl.pallas_call(
    _fin_body,
    grid=(N // BN_F,),
    in_specs=[pl.BlockSpec((4, BN_F, D), lambda i: (0, i, 0)),
              pl.BlockSpec((4, 128), lambda i: (0, 0))],
    out_specs=[pl.BlockSpec((BN_F, D), lambda i: (i, 0)),
               pl.BlockSpec((BN_F, D), lambda i: (i, 0))],
    out_shape=[jax.ShapeDtypeStruct((N, D), jnp.float32),
               jax.ShapeDtypeStruct((N, D), jnp.float32)],
)


def kernel(feat_user, feat_item, edge_index_u1, edge_index_u2,
           edge_index_i1, edge_index_i2, attn_l, attn_r,
           sem_W1, sem_b1, sem_W2):
    wu = jnp.stack([attn_r[0], attn_r[1], attn_l[2], attn_l[3]], axis=1)
    wi = jnp.stack([attn_l[0], attn_l[1], attn_r[2], attn_r[3]], axis=1)
    pu, pi = _proj(feat_user, feat_item, wu, wi)
    el_all = jnp.stack([pi[:, 0], pi[:, 1], pu[:, 2], pu[:, 3]]).reshape(-1)
    er_all = jnp.stack([pu[:, 0], pu[:, 1], pi[:, 2], pi[:, 3]]).reshape(-1)
    src_all = jnp.stack([edge_index_u1[0], edge_index_u2[0],
                         edge_index_i1[0], edge_index_i2[0]]).reshape(-1)
    dst_all = jnp.stack([edge_index_u1[1], edge_index_u2[1],
                         edge_index_i1[1], edge_index_i2[1]]).reshape(-1)
    feat_cat = jnp.concatenate([feat_item, feat_user], axis=0)
    out_part = _sc_edge(el_all, er_all, src_all, dst_all, feat_cat)
    h, w = _comb(out_part, sem_W1, sem_b1.reshape(1, HID),
                 sem_W2.reshape(1, HID))
    emb_u, emb_i = _fin(h, w)
    return (emb_u, emb_i)


# speedup vs baseline: 1.5976x; 1.5976x over previous
import functools

import jax
import jax.numpy as jnp
from jax import lax
from jax.experimental import pallas as pl
from jax.experimental.pallas import tpu as pltpu
from jax.experimental.pallas import tpu_sc as plsc

N = 10000
D = 128
E = 320000
HID = 128

NC = 2
NS = 16
L = 16

EPT = E // NS
CH = 800
NCH = EPT // CH
RCHUNK = 80
NSUB = CH // RCHUNK
NPAD = 10240
ROWS_T = NPAD // NS
ZROWS = 128


def _sc_edge_body(el_hbm, er_hbm, src_hbm, dst_hbm, feat_hbm,
                  out_hbm, ee_hbm,
                  a_v, b_v, isrc_c, idst_c, ee_c, a_c, csrc0, cdst0,
                  csrc1, cdst1, rows0, rows1, shared_s, shared_out,
                  gsem0, gsem1):
    cid = lax.axis_index("c")
    sid = lax.axis_index("s")

    for lp in range(2):
        p = cid * 2 + lp
        ebase = p * E + sid * EPT

        def _za(i, c):
            a_v[pl.ds(i * 16, 16)] = jnp.zeros((16,), jnp.float32)
            return c
        lax.fori_loop(0, N // 16, _za, 0)

        @pl.when(sid == 0)
        def _():
            pltpu.sync_copy(a_v, shared_s)

        def _zr(i, c):
            for j in range(8):
                rows0[i, pl.ds(j * 16, 16)] = jnp.zeros((16,), jnp.float32)
            return c
        lax.fori_loop(0, RCHUNK, _zr, 0)
        for k in range(ROWS_T // RCHUNK):
            pltpu.sync_copy(
                rows0,
                shared_out.at[pl.ds(sid * ROWS_T + k * RCHUNK, RCHUNK)])

        pltpu.sync_copy(el_hbm.at[pl.ds(p * N, N)], a_v)
        pltpu.sync_copy(er_hbm.at[pl.ds(p * N, N)], b_v)
        plsc.subcore_barrier()

        def _sch(ci, c):
            off = ebase + ci * CH
            pltpu.sync_copy(src_hbm.at[pl.ds(off, CH)], isrc_c)
            pltpu.sync_copy(dst_hbm.at[pl.ds(off, CH)], idst_c)

            def _sg(k, c2):
                s16 = isrc_c[pl.ds(k * 16, 16)]
                d16 = idst_c[pl.ds(k * 16, 16)]
                ev = (plsc.load_gather(a_v, [s16])
                      + plsc.load_gather(b_v, [d16]))
                ev = jnp.where(ev >= 0.0, ev, 0.01 * ev)
                ee_c[pl.ds(k * 16, 16)] = jnp.exp(ev)
                return c2
            lax.fori_loop(0, CH // 16, _sg, 0)
            pltpu.sync_copy(ee_c, ee_hbm.at[pl.ds(off, CH)])

            def _ssc(si, c2):
                soff = si * RCHUNK
                for k in range(RCHUNK // 16):
                    cdst0[pl.ds(k * 16, 16)] = idst_c[pl.ds(soff + k * 16, 16)]
                pltpu.sync_copy(ee_c.at[pl.ds(soff, RCHUNK)],
                                shared_s.at[cdst0], add=True)
                return c2
            lax.fori_loop(0, CH // RCHUNK, _ssc, 0)
            return c
        lax.fori_loop(0, NCH, _sch, 0)

        plsc.subcore_barrier()
        pltpu.sync_copy(shared_s, a_v)

        def _rcp(i, c):
            sl = pl.ds(i * 16, 16)
            a_v[sl] = 1.0 / (a_v[sl] + 1e-9)
            return c
        lax.fori_loop(0, N // 16, _rcp, 0)

        fbase = cid * N

        def _rch(ci, c):
            off = ebase + ci * CH
            pltpu.sync_copy(src_hbm.at[pl.ds(off, CH)], isrc_c)
            pltpu.sync_copy(dst_hbm.at[pl.ds(off, CH)], idst_c)
            pltpu.sync_copy(ee_hbm.at[pl.ds(off, CH)], ee_c)

            def _ag(k, c2):
                sl = pl.ds(k * 16, 16)
                d16 = idst_c[sl]
                a_c[sl] = ee_c[sl] * plsc.load_gather(a_v, [d16])
                return c2
            lax.fori_loop(0, CH // 16, _ag, 0)

            def _mk(soff, cs, cd):
                for k in range(RCHUNK // 16):
                    sl = pl.ds(k * 16, 16)
                    cs[sl] = isrc_c[pl.ds(soff + k * 16, 16)] + fbase
                    cd[sl] = idst_c[pl.ds(soff + k * 16, 16)]

            def _scale(soff, rows):
                def _sc(i, c3):
                    asp = plsc.load_gather(
                        a_c, [jnp.full((16,), soff + i, jnp.int32)])
                    for j in range(8):
                        sl = pl.ds(j * 16, 16)
                        rows[i, sl] = rows[i, sl] * asp
                    return c3
                lax.fori_loop(0, RCHUNK, _sc, 0)

            def _pair(hi, c2):
                soff0 = 2 * hi * RCHUNK
                _mk(soff0, csrc0, cdst0)
                pltpu.async_copy(feat_hbm.at[csrc0], rows0, gsem0).wait()
                _mk(soff0 + RCHUNK, csrc1, cdst1)
                cp1 = pltpu.async_copy(feat_hbm.at[csrc1], rows1, gsem1)
                _scale(soff0, rows0)
                pltpu.sync_copy(rows0, shared_out.at[cdst0], add=True)
                cp1.wait()
                _scale(soff0 + RCHUNK, rows1)
                pltpu.sync_copy(rows1, shared_out.at[cdst1], add=True)
                return c2
            lax.fori_loop(0, (CH // RCHUNK) // 2, _pair, 0)
            return c
        lax.fori_loop(0, NCH, _rch, 0)

        plsc.subcore_barrier()
        for k in range(ROWS_T // ZROWS):
            r0 = sid * ROWS_T + k * ZROWS
            pltpu.sync_copy(shared_out.at[pl.ds(r0, ZROWS)],
                            out_hbm.at[p, pl.ds(r0, ZROWS)])
        plsc.subcore_barrier()


_sc_edge = functools.partial(
    pl.kernel,
    out_type=(jax.ShapeDtypeStruct((4, NPAD, D), jnp.float32),
              jax.ShapeDtypeStruct((4 * E,), jnp.float32)),
    mesh=plsc.VectorSubcoreMesh(core_axis_name="c", subcore_axis_name="s"),
    compiler_params=pltpu.CompilerParams(needs_layout_passes=False),
    scratch_types=[
        pltpu.VMEM((N,), jnp.float32),
        pltpu.VMEM((N,), jnp.float32),
        pltpu.VMEM((CH,), jnp.int32),
        pltpu.VMEM((CH,), jnp.int32),
        pltpu.VMEM((CH,), jnp.float32),
        pltpu.VMEM((CH,), jnp.float32),
        pltpu.VMEM((RCHUNK,), jnp.int32),
        pltpu.VMEM((RCHUNK,), jnp.int32),
        pltpu.VMEM((RCHUNK,), jnp.int32),
        pltpu.VMEM((RCHUNK,), jnp.int32),
        pltpu.VMEM((RCHUNK, D), jnp.float32),
        pltpu.VMEM((RCHUNK, D), jnp.float32),
        pltpu.VMEM_SHARED((N,), jnp.float32),
        pltpu.VMEM_SHARED((NPAD, D), jnp.float32),
        pltpu.SemaphoreType.DMA,
        pltpu.SemaphoreType.DMA,
    ],
)(_sc_edge_body)


BN_A = 2000


def _proj_body(fu_ref, fi_ref, wu_ref, wi_ref, pu_ref, pi_ref):
    pu_ref[...] = jnp.dot(fu_ref[...], wu_ref[...],
                          preferred_element_type=jnp.float32)
    pi_ref[...] = jnp.dot(fi_ref[...], wi_ref[...],
                          preferred_element_type=jnp.float32)


_proj = pl.pallas_call(
    _proj_body,
    grid=(N // BN_A,),
    in_specs=[pl.BlockSpec((BN_A, D), lambda i: (i, 0)),
              pl.BlockSpec((BN_A, D), lambda i: (i, 0)),
              pl.BlockSpec((D, 4), lambda i: (0, 0)),
              pl.BlockSpec((D, 4), lambda i: (0, 0))],
    out_specs=[pl.BlockSpec((BN_A, 4), lambda i: (i, 0)),
               pl.BlockSpec((BN_A, 4), lambda i: (i, 0))],
    out_shape=[jax.ShapeDtypeStruct((N, 4), jnp.float32),
               jax.ShapeDtypeStruct((N, 4), jnp.float32)],
)

BN_B = 1000


def _comb_body(part_ref, w1_ref, b1_ref, w2_ref, h_ref, w_ref):
    i = pl.program_id(0)

    @pl.when(i == 0)
    def _():
        w_ref[...] = jnp.zeros_like(w_ref)

    for p in range(4):
        x = part_ref[p]
        h = jnp.where(x > 0.0, x, jnp.exp(x) - 1.0)
        h_ref[p] = h
        t = jnp.tanh(jnp.dot(h, w1_ref[...],
                             preferred_element_type=jnp.float32)
                     + b1_ref[...])
        contrib = jnp.sum(t * w2_ref[...])
        w_ref[pl.ds(p, 1), :] = w_ref[pl.ds(p, 1), :] + contrib


_comb = pl.pallas_call(
    _comb_body,
    grid=(N // BN_B,),
    in_specs=[pl.BlockSpec((4, BN_B, D), lambda i: (0, i, 0)),
              pl.BlockSpec((D, HID), lambda i: (0, 0)),
              pl.BlockSpec((1, HID), lambda i: (0, 0)),
              pl.BlockSpec((1, HID), lambda i: (0, 0))],
    out_specs=[pl.BlockSpec((4, BN_B, D), lambda i: (0, i, 0)),
               pl.BlockSpec((4, 128), lambda i: (0, 0))],
    out_shape=[jax.ShapeDtypeStruct((4, N, D), jnp.float32),
               jax.ShapeDtypeStruct((4, 128), jnp.float32)],
)

BN_F = 2000


def _fin_body(h_ref, w_ref, u_ref, i_ref):
    wv = w_ref[...] * (1.0 / N)
    wu0, wu1 = wv[0:1], wv[1:2]
    mu = jnp.maximum(wu0, wu1)
    e0, e1 = jnp.exp(wu0 - mu), jnp.exp(wu1 - mu)
    u_ref[...] = h_ref[0] * (e0 / (e0 + e1)) + h_ref[1] * (e1 / (e0 + e1))
    wi0, wi1 = wv[2:3], wv[3:4]
    mi = jnp.maximum(wi0, wi1)
    f0, f1 = jnp.exp(wi0 - mi), jnp.exp(wi1 - mi)
    i_ref[...] = h_ref[2] * (f0 / (f0 + f1)) + h_ref[3] * (f1 / (f0 + f1))


_fin = pl.pallas_call(
    _fin_body,
    grid=(N // BN_F,),
    in_specs=[pl.BlockSpec((4, BN_F, D), lambda i: (0, i, 0)),
              pl.BlockSpec((4, 128), lambda i: (0, 0))],
    out_specs=[pl.BlockSpec((BN_F, D), lambda i: (i, 0)),
               pl.BlockSpec((BN_F, D), lambda i: (i, 0))],
    out_shape=[jax.ShapeDtypeStruct((N, D), jnp.float32),
               jax.ShapeDtypeStruct((N, D), jnp.float32)],
)


def kernel(feat_user, feat_item, edge_index_u1, edge_index_u2,
           edge_index_i1, edge_index_i2, attn_l, attn_r,
           sem_W1, sem_b1, sem_W2):
    wu = jnp.stack([attn_r[0], attn_r[1], attn_l[2], attn_l[3]], axis=1)
    wi = jnp.stack([attn_l[0], attn_l[1], attn_r[2], attn_r[3]], axis=1)
    pu, pi = _proj(feat_user, feat_item, wu, wi)
    el_all = jnp.stack([pi[:, 0], pi[:, 1], pu[:, 2], pu[:, 3]]).reshape(-1)
    er_all = jnp.stack([pu[:, 0], pu[:, 1], pi[:, 2], pi[:, 3]]).reshape(-1)
    src_all = jnp.stack([edge_index_u1[0], edge_index_u2[0],
                         edge_index_i1[0], edge_index_i2[0]]).reshape(-1)
    dst_all = jnp.stack([edge_index_u1[1], edge_index_u2[1],
                         edge_index_i1[1], edge_index_i2[1]]).reshape(-1)
    feat_cat = jnp.concatenate([feat_item, feat_user], axis=0)
    out_part, _ = _sc_edge(el_all, er_all, src_all, dst_all, feat_cat)
    h, w = _comb(out_part, sem_W1, sem_b1.reshape(1, HID),
                 sem_W2.reshape(1, HID))
    emb_u, emb_i = _fin(h, w)
    return (emb_u, emb_i)


# speedup vs baseline: 1.9183x; 1.2008x over previous
import functools

import jax
import jax.numpy as jnp
from jax import lax
from jax.experimental import pallas as pl
from jax.experimental.pallas import tpu as pltpu
from jax.experimental.pallas import tpu_sc as plsc

N = 10000
D = 128
E = 320000
HID = 128

NC = 2
NS = 16
L = 16

EPT = E // NS
CH = 800
NCH = EPT // CH
RCHUNK = 80
NSUB = CH // RCHUNK
NPAD = 10240
ROWS_T = NPAD // NS
ZROWS = 128


def _sc_edge_body(el_hbm, er_hbm, src_hbm, dst_hbm, feat_hbm,
                  out_hbm, ee_hbm,
                  a_v, b_v, isrc_c, idst_c, ee_c, a_c, csrc0, cdst0,
                  csrc1, cdst1, rows0, rows1, shared_s, shared_out,
                  gsem0, gsem1):
    cid = lax.axis_index("c")
    sid = lax.axis_index("s")

    for lp in range(2):
        p = cid * 2 + lp
        ebase = p * E + sid * EPT

        def _za(i, c):
            a_v[pl.ds(i * 16, 16)] = jnp.zeros((16,), jnp.float32)
            return c
        lax.fori_loop(0, N // 16, _za, 0)

        @pl.when(sid == 0)
        def _():
            pltpu.sync_copy(a_v, shared_s)

        def _zr(i, c):
            for j in range(8):
                rows0[i, pl.ds(j * 16, 16)] = jnp.zeros((16,), jnp.float32)
            return c
        lax.fori_loop(0, RCHUNK, _zr, 0)
        for k in range(ROWS_T // RCHUNK):
            pltpu.sync_copy(
                rows0,
                shared_out.at[pl.ds(sid * ROWS_T + k * RCHUNK, RCHUNK)])

        pltpu.sync_copy(el_hbm.at[pl.ds(p * N, N)], a_v)
        pltpu.sync_copy(er_hbm.at[pl.ds(p * N, N)], b_v)
        plsc.subcore_barrier()

        def _sch(ci, c):
            off = ebase + ci * CH
            pltpu.sync_copy(src_hbm.at[pl.ds(off, CH)], isrc_c)
            pltpu.sync_copy(dst_hbm.at[pl.ds(off, CH)], idst_c)

            def _sg(k, c2):
                s16 = isrc_c[pl.ds(k * 16, 16)]
                d16 = idst_c[pl.ds(k * 16, 16)]
                ev = (plsc.load_gather(a_v, [s16])
                      + plsc.load_gather(b_v, [d16]))
                ev = jnp.where(ev >= 0.0, ev, 0.01 * ev)
                ee_c[pl.ds(k * 16, 16)] = jnp.exp(ev)
                return c2
            lax.fori_loop(0, CH // 16, _sg, 0)
            pltpu.sync_copy(ee_c, ee_hbm.at[pl.ds(off, CH)])

            def _ssc(si, c2):
                soff = si * RCHUNK
                for k in range(RCHUNK // 16):
                    cdst0[pl.ds(k * 16, 16)] = idst_c[pl.ds(soff + k * 16, 16)]
                pltpu.sync_copy(ee_c.at[pl.ds(soff, RCHUNK)],
                                shared_s.at[cdst0], add=True)
                return c2
            lax.fori_loop(0, CH // RCHUNK, _ssc, 0)
            return c
        lax.fori_loop(0, NCH, _sch, 0)

        plsc.subcore_barrier()
        pltpu.sync_copy(shared_s, a_v)

        def _rcp(i, c):
            sl = pl.ds(i * 16, 16)
            a_v[sl] = 1.0 / (a_v[sl] + 1e-9)
            return c
        lax.fori_loop(0, N // 16, _rcp, 0)

        fbase = cid * N

        def _rch(ci, c):
            off = ebase + ci * CH
            pltpu.sync_copy(src_hbm.at[pl.ds(off, CH)], isrc_c)
            pltpu.sync_copy(dst_hbm.at[pl.ds(off, CH)], idst_c)
            pltpu.sync_copy(ee_hbm.at[pl.ds(off, CH)], ee_c)

            def _ag(k, c2):
                sl = pl.ds(k * 16, 16)
                d16 = idst_c[sl]
                a_c[sl] = ee_c[sl] * plsc.load_gather(a_v, [d16])
                return c2
            lax.fori_loop(0, CH // 16, _ag, 0)

            def _mk(soff, cs, cd):
                for k in range(RCHUNK // 16):
                    sl = pl.ds(k * 16, 16)
                    cs[sl] = isrc_c[pl.ds(soff + k * 16, 16)] + fbase
                    cd[sl] = idst_c[pl.ds(soff + k * 16, 16)]

            def _scale(soff, rows):
                def _sc(i, c3):
                    asp = plsc.load_gather(
                        a_c, [jnp.full((16,), soff + i, jnp.int32)])
                    for j in range(8):
                        sl = pl.ds(j * 16, 16)
                        rows[i, sl] = rows[i, sl] * asp
                    return c3
                lax.fori_loop(0, RCHUNK, _sc, 0)

            bufs = [(csrc0, cdst0, rows0, gsem0),
                    (csrc1, cdst1, rows1, gsem1)]
            nsub = CH // RCHUNK
            _mk(0, csrc0, cdst0)
            cp = pltpu.async_copy(feat_hbm.at[csrc0], rows0, gsem0)
            for si in range(nsub):
                cs, cd, rows, _ = bufs[si % 2]
                cp.wait()
                if si + 1 < nsub:
                    ns, nd, nrows, nsem = bufs[(si + 1) % 2]
                    _mk((si + 1) * RCHUNK, ns, nd)
                    cp = pltpu.async_copy(feat_hbm.at[ns], nrows, nsem)
                _scale(si * RCHUNK, rows)
                pltpu.sync_copy(rows, shared_out.at[cd], add=True)
            return c
        lax.fori_loop(0, NCH, _rch, 0)

        plsc.subcore_barrier()
        for k in range(ROWS_T // ZROWS):
            r0 = sid * ROWS_T + k * ZROWS
            pltpu.sync_copy(shared_out.at[pl.ds(r0, ZROWS)],
                            out_hbm.at[p, pl.ds(r0, ZROWS)])
        plsc.subcore_barrier()


_sc_edge = functools.partial(
    pl.kernel,
    out_type=(jax.ShapeDtypeStruct((4, NPAD, D), jnp.float32),
              jax.ShapeDtypeStruct((4 * E,), jnp.float32)),
    mesh=plsc.VectorSubcoreMesh(core_axis_name="c", subcore_axis_name="s"),
    compiler_params=pltpu.CompilerParams(needs_layout_passes=False),
    scratch_types=[
        pltpu.VMEM((N,), jnp.float32),
        pltpu.VMEM((N,), jnp.float32),
        pltpu.VMEM((CH,), jnp.int32),
        pltpu.VMEM((CH,), jnp.int32),
        pltpu.VMEM((CH,), jnp.float32),
        pltpu.VMEM((CH,), jnp.float32),
        pltpu.VMEM((RCHUNK,), jnp.int32),
        pltpu.VMEM((RCHUNK,), jnp.int32),
        pltpu.VMEM((RCHUNK,), jnp.int32),
        pltpu.VMEM((RCHUNK,), jnp.int32),
        pltpu.VMEM((RCHUNK, D), jnp.float32),
        pltpu.VMEM((RCHUNK, D), jnp.float32),
        pltpu.VMEM_SHARED((N,), jnp.float32),
        pltpu.VMEM_SHARED((NPAD, D), jnp.float32),
        pltpu.SemaphoreType.DMA,
        pltpu.SemaphoreType.DMA,
    ],
)(_sc_edge_body)


BN_A = 2000


def _proj_body(fu_ref, fi_ref, wu_ref, wi_ref, pu_ref, pi_ref):
    pu_ref[...] = jnp.dot(fu_ref[...], wu_ref[...],
                          preferred_element_type=jnp.float32)
    pi_ref[...] = jnp.dot(fi_ref[...], wi_ref[...],
                          preferred_element_type=jnp.float32)


_proj = pl.pallas_call(
    _proj_body,
    grid=(N // BN_A,),
    in_specs=[pl.BlockSpec((BN_A, D), lambda i: (i, 0)),
              pl.BlockSpec((BN_A, D), lambda i: (i, 0)),
              pl.BlockSpec((D, 4), lambda i: (0, 0)),
              pl.BlockSpec((D, 4), lambda i: (0, 0))],
    out_specs=[pl.BlockSpec((BN_A, 4), lambda i: (i, 0)),
               pl.BlockSpec((BN_A, 4), lambda i: (i, 0))],
    out_shape=[jax.ShapeDtypeStruct((N, 4), jnp.float32),
               jax.ShapeDtypeStruct((N, 4), jnp.float32)],
)

BN_B = 1000


def _comb_body(part_ref, w1_ref, b1_ref, w2_ref, h_ref, w_ref):
    i = pl.program_id(0)

    @pl.when(i == 0)
    def _():
        w_ref[...] = jnp.zeros_like(w_ref)

    for p in range(4):
        x = part_ref[p]
        h = jnp.where(x > 0.0, x, jnp.exp(x) - 1.0)
        h_ref[p] = h
        t = jnp.tanh(jnp.dot(h, w1_ref[...],
                             preferred_element_type=jnp.float32)
                     + b1_ref[...])
        contrib = jnp.sum(t * w2_ref[...])
        w_ref[pl.ds(p, 1), :] = w_ref[pl.ds(p, 1), :] + contrib


_comb = pl.pallas_call(
    _comb_body,
    grid=(N // BN_B,),
    in_specs=[pl.BlockSpec((4, BN_B, D), lambda i: (0, i, 0)),
              pl.BlockSpec((D, HID), lambda i: (0, 0)),
              pl.BlockSpec((1, HID), lambda i: (0, 0)),
              pl.BlockSpec((1, HID), lambda i: (0, 0))],
    out_specs=[pl.BlockSpec((4, BN_B, D), lambda i: (0, i, 0)),
               pl.BlockSpec((4, 128), lambda i: (0, 0))],
    out_shape=[jax.ShapeDtypeStruct((4, N, D), jnp.float32),
               jax.ShapeDtypeStruct((4, 128), jnp.float32)],
)

BN_F = 2000


def _fin_body(h_ref, w_ref, u_ref, i_ref):
    wv = w_ref[...] * (1.0 / N)
    wu0, wu1 = wv[0:1], wv[1:2]
    mu = jnp.maximum(wu0, wu1)
    e0, e1 = jnp.exp(wu0 - mu), jnp.exp(wu1 - mu)
    u_ref[...] = h_ref[0] * (e0 / (e0 + e1)) + h_ref[1] * (e1 / (e0 + e1))
    wi0, wi1 = wv[2:3], wv[3:4]
    mi = jnp.maximum(wi0, wi1)
    f0, f1 = jnp.exp(wi0 - mi), jnp.exp(wi1 - mi)
    i_ref[...] = h_ref[2] * (f0 / (f0 + f1)) + h_ref[3] * (f1 / (f0 + f1))


_fin = pl.pallas_call(
    _fin_body,
    grid=(N // BN_F,),
    in_specs=[pl.BlockSpec((4, BN_F, D), lambda i: (0, i, 0)),
              pl.BlockSpec((4, 128), lambda i: (0, 0))],
    out_specs=[pl.BlockSpec((BN_F, D), lambda i: (i, 0)),
               pl.BlockSpec((BN_F, D), lambda i: (i, 0))],
    out_shape=[jax.ShapeDtypeStruct((N, D), jnp.float32),
               jax.ShapeDtypeStruct((N, D), jnp.float32)],
)


def kernel(feat_user, feat_item, edge_index_u1, edge_index_u2,
           edge_index_i1, edge_index_i2, attn_l, attn_r,
           sem_W1, sem_b1, sem_W2):
    wu = jnp.stack([attn_r[0], attn_r[1], attn_l[2], attn_l[3]], axis=1)
    wi = jnp.stack([attn_l[0], attn_l[1], attn_r[2], attn_r[3]], axis=1)
    pu, pi = _proj(feat_user, feat_item, wu, wi)
    el_all = jnp.stack([pi[:, 0], pi[:, 1], pu[:, 2], pu[:, 3]]).reshape(-1)
    er_all = jnp.stack([pu[:, 0], pu[:, 1], pi[:, 2], pi[:, 3]]).reshape(-1)
    src_all = jnp.stack([edge_index_u1[0], edge_index_u2[0],
                         edge_index_i1[0], edge_index_i2[0]]).reshape(-1)
    dst_all = jnp.stack([edge_index_u1[1], edge_index_u2[1],
                         edge_index_i1[1], edge_index_i2[1]]).reshape(-1)
    feat_cat = jnp.concatenate([feat_item, feat_user], axis=0)
    out_part, _ = _sc_edge(el_all, er_all, src_all, dst_all, feat_cat)
    h, w = _comb(out_part, sem_W1, sem_b1.reshape(1, HID),
                 sem_W2.reshape(1, HID))
    emb_u, emb_i = _fin(h, w)
    return (emb_u, emb_i)


# speedup vs baseline: 1.9349x; 1.0087x over previous
import functools

import jax
import jax.numpy as jnp
from jax import lax
from jax.experimental import pallas as pl
from jax.experimental.pallas import tpu as pltpu
from jax.experimental.pallas import tpu_sc as plsc

N = 10000
D = 128
E = 320000
HID = 128

NC = 2
NS = 16
L = 16

EPT = E // NS
CH = 800
NCH = EPT // CH
RCHUNK = 80
NSUB = CH // RCHUNK
NPAD = 10240
ROWS_T = NPAD // NS
ZROWS = 128


def _sc_edge_body(el_hbm, er_hbm, src_hbm, dst_hbm, feat_hbm,
                  out_hbm, ee_hbm,
                  a_v, b_v, isrc_c, idst_c, ee_c, a_c, csrc0, cdst0,
                  csrc1, cdst1, rows0, rows1, shared_s, shared_out,
                  gsem0, gsem1, ssem0, ssem1):
    cid = lax.axis_index("c")
    sid = lax.axis_index("s")

    for lp in range(2):
        p = cid * 2 + lp
        ebase = p * E + sid * EPT

        def _za(i, c):
            a_v[pl.ds(i * 16, 16)] = jnp.zeros((16,), jnp.float32)
            return c
        lax.fori_loop(0, N // 16, _za, 0)

        @pl.when(sid == 0)
        def _():
            pltpu.sync_copy(a_v, shared_s)

        def _zr(i, c):
            for j in range(8):
                rows0[i, pl.ds(j * 16, 16)] = jnp.zeros((16,), jnp.float32)
            return c
        lax.fori_loop(0, RCHUNK, _zr, 0)
        for k in range(ROWS_T // RCHUNK):
            pltpu.sync_copy(
                rows0,
                shared_out.at[pl.ds(sid * ROWS_T + k * RCHUNK, RCHUNK)])

        pltpu.sync_copy(el_hbm.at[pl.ds(p * N, N)], a_v)
        pltpu.sync_copy(er_hbm.at[pl.ds(p * N, N)], b_v)
        plsc.subcore_barrier()

        def _sch(ci, c):
            off = ebase + ci * CH
            pltpu.sync_copy(src_hbm.at[pl.ds(off, CH)], isrc_c)
            pltpu.sync_copy(dst_hbm.at[pl.ds(off, CH)], idst_c)

            def _sg(k, c2):
                s16 = isrc_c[pl.ds(k * 16, 16)]
                d16 = idst_c[pl.ds(k * 16, 16)]
                ev = (plsc.load_gather(a_v, [s16])
                      + plsc.load_gather(b_v, [d16]))
                ev = jnp.where(ev >= 0.0, ev, 0.01 * ev)
                ee_c[pl.ds(k * 16, 16)] = jnp.exp(ev)
                return c2
            lax.fori_loop(0, CH // 16, _sg, 0)
            pltpu.sync_copy(ee_c, ee_hbm.at[pl.ds(off, CH)])

            def _ssc(si, c2):
                soff = si * RCHUNK
                for k in range(RCHUNK // 16):
                    cdst0[pl.ds(k * 16, 16)] = idst_c[pl.ds(soff + k * 16, 16)]
                pltpu.sync_copy(ee_c.at[pl.ds(soff, RCHUNK)],
                                shared_s.at[cdst0], add=True)
                return c2
            lax.fori_loop(0, CH // RCHUNK, _ssc, 0)
            return c
        lax.fori_loop(0, NCH, _sch, 0)

        plsc.subcore_barrier()
        pltpu.sync_copy(shared_s, a_v)

        def _rcp(i, c):
            sl = pl.ds(i * 16, 16)
            a_v[sl] = 1.0 / (a_v[sl] + 1e-9)
            return c
        lax.fori_loop(0, N // 16, _rcp, 0)

        fbase = cid * N

        def _rch(ci, c):
            off = ebase + ci * CH
            pltpu.sync_copy(src_hbm.at[pl.ds(off, CH)], isrc_c)
            pltpu.sync_copy(dst_hbm.at[pl.ds(off, CH)], idst_c)
            pltpu.sync_copy(ee_hbm.at[pl.ds(off, CH)], ee_c)

            def _ag(k, c2):
                sl = pl.ds(k * 16, 16)
                d16 = idst_c[sl]
                a_c[sl] = ee_c[sl] * plsc.load_gather(a_v, [d16])
                return c2
            lax.fori_loop(0, CH // 16, _ag, 0)

            def _mk(soff, cs, cd):
                for k in range(RCHUNK // 16):
                    sl = pl.ds(k * 16, 16)
                    cs[sl] = isrc_c[pl.ds(soff + k * 16, 16)] + fbase
                    cd[sl] = idst_c[pl.ds(soff + k * 16, 16)]

            def _scale(soff, rows):
                def _sc(i, c3):
                    asp = plsc.load_gather(
                        a_c, [jnp.full((16,), soff + i, jnp.int32)])
                    for j in range(8):
                        sl = pl.ds(j * 16, 16)
                        rows[i, sl] = rows[i, sl] * asp
                    return c3
                lax.fori_loop(0, RCHUNK, _sc, 0)

            bufs = [(csrc0, cdst0, rows0, gsem0, ssem0),
                    (csrc1, cdst1, rows1, gsem1, ssem1)]
            nsub = CH // RCHUNK
            scp = [None, None]
            _mk(0, csrc0, cdst0)
            cp = pltpu.async_copy(feat_hbm.at[csrc0], rows0, gsem0)
            for si in range(nsub):
                cs, cd, rows, _, ssem = bufs[si % 2]
                cp.wait()
                if si + 1 < nsub:
                    ns, nd, nrows, nsem, _ = bufs[(si + 1) % 2]
                    if scp[(si + 1) % 2] is not None:
                        scp[(si + 1) % 2].wait()
                    _mk((si + 1) * RCHUNK, ns, nd)
                    cp = pltpu.async_copy(feat_hbm.at[ns], nrows, nsem)
                _scale(si * RCHUNK, rows)
                scp[si % 2] = pltpu.async_copy(
                    rows, shared_out.at[cd], ssem, add=True)
            scp[(nsub - 1) % 2].wait()
            scp[nsub % 2].wait()
            return c
        lax.fori_loop(0, NCH, _rch, 0)

        plsc.subcore_barrier()
        for k in range(ROWS_T // ZROWS):
            r0 = sid * ROWS_T + k * ZROWS
            pltpu.sync_copy(shared_out.at[pl.ds(r0, ZROWS)],
                            out_hbm.at[p, pl.ds(r0, ZROWS)])
        plsc.subcore_barrier()


_sc_edge = functools.partial(
    pl.kernel,
    out_type=(jax.ShapeDtypeStruct((4, NPAD, D), jnp.float32),
              jax.ShapeDtypeStruct((4 * E,), jnp.float32)),
    mesh=plsc.VectorSubcoreMesh(core_axis_name="c", subcore_axis_name="s"),
    compiler_params=pltpu.CompilerParams(needs_layout_passes=False),
    scratch_types=[
        pltpu.VMEM((N,), jnp.float32),
        pltpu.VMEM((N,), jnp.float32),
        pltpu.VMEM((CH,), jnp.int32),
        pltpu.VMEM((CH,), jnp.int32),
        pltpu.VMEM((CH,), jnp.float32),
        pltpu.VMEM((CH,), jnp.float32),
        pltpu.VMEM((RCHUNK,), jnp.int32),
        pltpu.VMEM((RCHUNK,), jnp.int32),
        pltpu.VMEM((RCHUNK,), jnp.int32),
        pltpu.VMEM((RCHUNK,), jnp.int32),
        pltpu.VMEM((RCHUNK, D), jnp.float32),
        pltpu.VMEM((RCHUNK, D), jnp.float32),
        pltpu.VMEM_SHARED((N,), jnp.float32),
        pltpu.VMEM_SHARED((NPAD, D), jnp.float32),
        pltpu.SemaphoreType.DMA,
        pltpu.SemaphoreType.DMA,
        pltpu.SemaphoreType.DMA,
        pltpu.SemaphoreType.DMA,
    ],
)(_sc_edge_body)


BN_A = 2000


def _proj_body(fu_ref, fi_ref, wu_ref, wi_ref, pu_ref, pi_ref):
    pu_ref[...] = jnp.dot(fu_ref[...], wu_ref[...],
                          preferred_element_type=jnp.float32)
    pi_ref[...] = jnp.dot(fi_ref[...], wi_ref[...],
                          preferred_element_type=jnp.float32)


_proj = pl.pallas_call(
    _proj_body,
    grid=(N // BN_A,),
    in_specs=[pl.BlockSpec((BN_A, D), lambda i: (i, 0)),
              pl.BlockSpec((BN_A, D), lambda i: (i, 0)),
              pl.BlockSpec((D, 4), lambda i: (0, 0)),
              pl.BlockSpec((D, 4), lambda i: (0, 0))],
    out_specs=[pl.BlockSpec((BN_A, 4), lambda i: (i, 0)),
               pl.BlockSpec((BN_A, 4), lambda i: (i, 0))],
    out_shape=[jax.ShapeDtypeStruct((N, 4), jnp.float32),
               jax.ShapeDtypeStruct((N, 4), jnp.float32)],
)

BN_B = 1000


def _comb_body(part_ref, w1_ref, b1_ref, w2_ref, h_ref, w_ref):
    i = pl.program_id(0)

    @pl.when(i == 0)
    def _():
        w_ref[...] = jnp.zeros_like(w_ref)

    for p in range(4):
        x = part_ref[p]
        h = jnp.where(x > 0.0, x, jnp.exp(x) - 1.0)
        h_ref[p] = h
        t = jnp.tanh(jnp.dot(h, w1_ref[...],
                             preferred_element_type=jnp.float32)
                     + b1_ref[...])
        contrib = jnp.sum(t * w2_ref[...])
        w_ref[pl.ds(p, 1), :] = w_ref[pl.ds(p, 1), :] + contrib


_comb = pl.pallas_call(
    _comb_body,
    grid=(N // BN_B,),
    in_specs=[pl.BlockSpec((4, BN_B, D), lambda i: (0, i, 0)),
              pl.BlockSpec((D, HID), lambda i: (0, 0)),
              pl.BlockSpec((1, HID), lambda i: (0, 0)),
              pl.BlockSpec((1, HID), lambda i: (0, 0))],
    out_specs=[pl.BlockSpec((4, BN_B, D), lambda i: (0, i, 0)),
               pl.BlockSpec((4, 128), lambda i: (0, 0))],
    out_shape=[jax.ShapeDtypeStruct((4, N, D), jnp.float32),
               jax.ShapeDtypeStruct((4, 128), jnp.float32)],
)

BN_F = 2000


def _fin_body(h_ref, w_ref, u_ref, i_ref):
    wv = w_ref[...] * (1.0 / N)
    wu0, wu1 = wv[0:1], wv[1:2]
    mu = jnp.maximum(wu0, wu1)
    e0, e1 = jnp.exp(wu0 - mu), jnp.exp(wu1 - mu)
    u_ref[...] = h_ref[0] * (e0 / (e0 + e1)) + h_ref[1] * (e1 / (e0 + e1))
    wi0, wi1 = wv[2:3], wv[3:4]
    mi = jnp.maximum(wi0, wi1)
    f0, f1 = jnp.exp(wi0 - mi), jnp.exp(wi1 - mi)
    i_ref[...] = h_ref[2] * (f0 / (f0 + f1)) + h_ref[3] * (f1 / (f0 + f1))


_fin = pl.pallas_call(
    _fin_body,
    grid=(N // BN_F,),
    in_specs=[pl.BlockSpec((4, BN_F, D), lambda i: (0, i, 0)),
              pl.BlockSpec((4, 128), lambda i: (0, 0))],
    out_specs=[pl.BlockSpec((BN_F, D), lambda i: (i, 0)),
               pl.BlockSpec((BN_F, D), lambda i: (i, 0))],
    out_shape=[jax.ShapeDtypeStruct((N, D), jnp.float32),
               jax.ShapeDtypeStruct((N, D), jnp.float32)],
)


def kernel(feat_user, feat_item, edge_index_u1, edge_index_u2,
           edge_index_i1, edge_index_i2, attn_l, attn_r,
           sem_W1, sem_b1, sem_W2):
    wu = jnp.stack([attn_r[0], attn_r[1], attn_l[2], attn_l[3]], axis=1)
    wi = jnp.stack([attn_l[0], attn_l[1], attn_r[2], attn_r[3]], axis=1)
    pu, pi = _proj(feat_user, feat_item, wu, wi)
    el_all = jnp.stack([pi[:, 0], pi[:, 1], pu[:, 2], pu[:, 3]]).reshape(-1)
    er_all = jnp.stack([pu[:, 0], pu[:, 1], pi[:, 2], pi[:, 3]]).reshape(-1)
    src_all = jnp.stack([edge_index_u1[0], edge_index_u2[0],
                         edge_index_i1[0], edge_index_i2[0]]).reshape(-1)
    dst_all = jnp.stack([edge_index_u1[1], edge_index_u2[1],
                         edge_index_i1[1], edge_index_i2[1]]).reshape(-1)
    feat_cat = jnp.concatenate([feat_item, feat_user], axis=0)
    out_part, _ = _sc_edge(el_all, er_all, src_all, dst_all, feat_cat)
    h, w = _comb(out_part, sem_W1, sem_b1.reshape(1, HID),
                 sem_W2.reshape(1, HID))
    emb_u, emb_i = _fin(h, w)
    return (emb_u, emb_i)


# speedup vs baseline: 1.9699x; 1.0180x over previous
import functools

import jax
import jax.numpy as jnp
from jax import lax
from jax.experimental import pallas as pl
from jax.experimental.pallas import tpu as pltpu
from jax.experimental.pallas import tpu_sc as plsc

N = 10000
D = 128
E = 320000
HID = 128

NC = 2
NS = 16
L = 16

EPT = E // NS
CH = 800
NCH = EPT // CH
RCHUNK = 80
NSUB = CH // RCHUNK
NPAD = 10240
ROWS_T = NPAD // NS
ZROWS = 128


def _sc_edge_body(el_hbm, er_hbm, src_hbm, dst_hbm, feat_hbm,
                  out_hbm, ee_hbm,
                  a_v, b_v, isrc_c, idst_c, ee_c, a_c, csrc0, cdst0,
                  csrc1, cdst1, rows0, rows1, shared_s, shared_out,
                  gsem0, gsem1, ssem0, ssem1):
    cid = lax.axis_index("c")
    sid = lax.axis_index("s")

    for lp in range(2):
        p = cid * 2 + lp
        ebase = p * E + sid * EPT

        def _za(i, c):
            a_v[pl.ds(i * 16, 16)] = jnp.zeros((16,), jnp.float32)
            return c
        lax.fori_loop(0, N // 16, _za, 0)

        @pl.when(sid == 0)
        def _():
            pltpu.sync_copy(a_v, shared_s)

        def _zr(i, c):
            for j in range(8):
                rows0[i, pl.ds(j * 16, 16)] = jnp.zeros((16,), jnp.float32)
            return c
        lax.fori_loop(0, RCHUNK, _zr, 0)
        for k in range(ROWS_T // RCHUNK):
            pltpu.sync_copy(
                rows0,
                shared_out.at[pl.ds(sid * ROWS_T + k * RCHUNK, RCHUNK)])

        pltpu.sync_copy(el_hbm.at[pl.ds(p * N, N)], a_v)
        pltpu.sync_copy(er_hbm.at[pl.ds(p * N, N)], b_v)
        plsc.subcore_barrier()

        def _sch(ci, c):
            off = ebase + ci * CH
            pltpu.sync_copy(src_hbm.at[pl.ds(off, CH)], isrc_c)
            pltpu.sync_copy(dst_hbm.at[pl.ds(off, CH)], idst_c)

            def _sg(k, c2):
                s16 = isrc_c[pl.ds(k * 16, 16)]
                d16 = idst_c[pl.ds(k * 16, 16)]
                ev = (plsc.load_gather(a_v, [s16])
                      + plsc.load_gather(b_v, [d16]))
                ev = jnp.where(ev >= 0.0, ev, 0.01 * ev)
                ee_c[pl.ds(k * 16, 16)] = jnp.exp(ev)
                return c2
            lax.fori_loop(0, CH // 16, _sg, 0)
            pltpu.sync_copy(ee_c, ee_hbm.at[pl.ds(off, CH)])

            sbufs = [(cdst0, ssem0), (cdst1, ssem1)]
            sscp = [None, None]
            for si in range(CH // RCHUNK):
                cdx, ssx = sbufs[si % 2]
                if sscp[si % 2] is not None:
                    sscp[si % 2].wait()
                soff = si * RCHUNK
                for k in range(RCHUNK // 16):
                    cdx[pl.ds(k * 16, 16)] = idst_c[pl.ds(soff + k * 16, 16)]
                sscp[si % 2] = pltpu.async_copy(
                    ee_c.at[pl.ds(soff, RCHUNK)], shared_s.at[cdx],
                    ssx, add=True)
            sscp[0].wait()
            sscp[1].wait()
            return c
        lax.fori_loop(0, NCH, _sch, 0)

        plsc.subcore_barrier()
        pltpu.sync_copy(shared_s, a_v)

        def _rcp(i, c):
            sl = pl.ds(i * 16, 16)
            a_v[sl] = 1.0 / (a_v[sl] + 1e-9)
            return c
        lax.fori_loop(0, N // 16, _rcp, 0)

        fbase = cid * N

        def _rch(ci, c):
            off = ebase + ci * CH
            pltpu.sync_copy(src_hbm.at[pl.ds(off, CH)], isrc_c)
            pltpu.sync_copy(dst_hbm.at[pl.ds(off, CH)], idst_c)
            pltpu.sync_copy(ee_hbm.at[pl.ds(off, CH)], ee_c)

            def _ag(k, c2):
                sl = pl.ds(k * 16, 16)
                d16 = idst_c[sl]
                a_c[sl] = ee_c[sl] * plsc.load_gather(a_v, [d16])
                return c2
            lax.fori_loop(0, CH // 16, _ag, 0)

            def _mk(soff, cs, cd):
                for k in range(RCHUNK // 16):
                    sl = pl.ds(k * 16, 16)
                    cs[sl] = isrc_c[pl.ds(soff + k * 16, 16)] + fbase
                    cd[sl] = idst_c[pl.ds(soff + k * 16, 16)]

            def _scale(soff, rows):
                def _sc(i, c3):
                    asp = plsc.load_gather(
                        a_c, [jnp.full((16,), soff + i, jnp.int32)])
                    for j in range(8):
                        sl = pl.ds(j * 16, 16)
                        rows[i, sl] = rows[i, sl] * asp
                    return c3
                lax.fori_loop(0, RCHUNK, _sc, 0)

            bufs = [(csrc0, cdst0, rows0, gsem0, ssem0),
                    (csrc1, cdst1, rows1, gsem1, ssem1)]
            nsub = CH // RCHUNK
            scp = [None, None]
            _mk(0, csrc0, cdst0)
            cp = pltpu.async_copy(feat_hbm.at[csrc0], rows0, gsem0)
            for si in range(nsub):
                cs, cd, rows, _, ssem = bufs[si % 2]
                cp.wait()
                if si + 1 < nsub:
                    ns, nd, nrows, nsem, _ = bufs[(si + 1) % 2]
                    if scp[(si + 1) % 2] is not None:
                        scp[(si + 1) % 2].wait()
                    _mk((si + 1) * RCHUNK, ns, nd)
                    cp = pltpu.async_copy(feat_hbm.at[ns], nrows, nsem)
                _scale(si * RCHUNK, rows)
                scp[si % 2] = pltpu.async_copy(
                    rows, shared_out.at[cd], ssem, add=True)
            scp[(nsub - 1) % 2].wait()
            scp[nsub % 2].wait()
            return c
        lax.fori_loop(0, NCH, _rch, 0)

        plsc.subcore_barrier()
        for k in range(ROWS_T // ZROWS):
            r0 = sid * ROWS_T + k * ZROWS
            pltpu.sync_copy(shared_out.at[pl.ds(r0, ZROWS)],
                            out_hbm.at[p, pl.ds(r0, ZROWS)])
        plsc.subcore_barrier()


_sc_edge = functools.partial(
    pl.kernel,
    out_type=(jax.ShapeDtypeStruct((4, NPAD, D), jnp.float32),
              jax.ShapeDtypeStruct((4 * E,), jnp.float32)),
    mesh=plsc.VectorSubcoreMesh(core_axis_name="c", subcore_axis_name="s"),
    compiler_params=pltpu.CompilerParams(needs_layout_passes=False),
    scratch_types=[
        pltpu.VMEM((N,), jnp.float32),
        pltpu.VMEM((N,), jnp.float32),
        pltpu.VMEM((CH,), jnp.int32),
        pltpu.VMEM((CH,), jnp.int32),
        pltpu.VMEM((CH,), jnp.float32),
        pltpu.VMEM((CH,), jnp.float32),
        pltpu.VMEM((RCHUNK,), jnp.int32),
        pltpu.VMEM((RCHUNK,), jnp.int32),
        pltpu.VMEM((RCHUNK,), jnp.int32),
        pltpu.VMEM((RCHUNK,), jnp.int32),
        pltpu.VMEM((RCHUNK, D), jnp.float32),
        pltpu.VMEM((RCHUNK, D), jnp.float32),
        pltpu.VMEM_SHARED((N,), jnp.float32),
        pltpu.VMEM_SHARED((NPAD, D), jnp.float32),
        pltpu.SemaphoreType.DMA,
        pltpu.SemaphoreType.DMA,
        pltpu.SemaphoreType.DMA,
        pltpu.SemaphoreType.DMA,
    ],
)(_sc_edge_body)


BN_A = 2000


def _proj_body(fu_ref, fi_ref, wu_ref, wi_ref, pu_ref, pi_ref):
    pu_ref[...] = jnp.dot(fu_ref[...], wu_ref[...],
                          preferred_element_type=jnp.float32)
    pi_ref[...] = jnp.dot(fi_ref[...], wi_ref[...],
                          preferred_element_type=jnp.float32)


_proj = pl.pallas_call(
    _proj_body,
    grid=(N // BN_A,),
    in_specs=[pl.BlockSpec((BN_A, D), lambda i: (i, 0)),
              pl.BlockSpec((BN_A, D), lambda i: (i, 0)),
              pl.BlockSpec((D, 4), lambda i: (0, 0)),
              pl.BlockSpec((D, 4), lambda i: (0, 0))],
    out_specs=[pl.BlockSpec((BN_A, 4), lambda i: (i, 0)),
               pl.BlockSpec((BN_A, 4), lambda i: (i, 0))],
    out_shape=[jax.ShapeDtypeStruct((N, 4), jnp.float32),
               jax.ShapeDtypeStruct((N, 4), jnp.float32)],
)

BN_B = 1000


def _comb_body(part_ref, w1_ref, b1_ref, w2_ref, h_ref, w_ref):
    i = pl.program_id(0)

    @pl.when(i == 0)
    def _():
        w_ref[...] = jnp.zeros_like(w_ref)

    for p in range(4):
        x = part_ref[p]
        h = jnp.where(x > 0.0, x, jnp.exp(x) - 1.0)
        h_ref[p] = h
        t = jnp.tanh(jnp.dot(h, w1_ref[...],
                             preferred_element_type=jnp.float32)
                     + b1_ref[...])
        contrib = jnp.sum(t * w2_ref[...])
        w_ref[pl.ds(p, 1), :] = w_ref[pl.ds(p, 1), :] + contrib


_comb = pl.pallas_call(
    _comb_body,
    grid=(N // BN_B,),
    in_specs=[pl.BlockSpec((4, BN_B, D), lambda i: (0, i, 0)),
              pl.BlockSpec((D, HID), lambda i: (0, 0)),
              pl.BlockSpec((1, HID), lambda i: (0, 0)),
              pl.BlockSpec((1, HID), lambda i: (0, 0))],
    out_specs=[pl.BlockSpec((4, BN_B, D), lambda i: (0, i, 0)),
               pl.BlockSpec((4, 128), lambda i: (0, 0))],
    out_shape=[jax.ShapeDtypeStruct((4, N, D), jnp.float32),
               jax.ShapeDtypeStruct((4, 128), jnp.float32)],
)

BN_F = 2000


def _fin_body(h_ref, w_ref, u_ref, i_ref):
    wv = w_ref[...] * (1.0 / N)
    wu0, wu1 = wv[0:1], wv[1:2]
    mu = jnp.maximum(wu0, wu1)
    e0, e1 = jnp.exp(wu0 - mu), jnp.exp(wu1 - mu)
    u_ref[...] = h_ref[0] * (e0 / (e0 + e1)) + h_ref[1] * (e1 / (e0 + e1))
    wi0, wi1 = wv[2:3], wv[3:4]
    mi = jnp.maximum(wi0, wi1)
    f0, f1 = jnp.exp(wi0 - mi), jnp.exp(wi1 - mi)
    i_ref[...] = h_ref[2] * (f0 / (f0 + f1)) + h_ref[3] * (f1 / (f0 + f1))


_fin = pl.pallas_call(
    _fin_body,
    grid=(N // BN_F,),
    in_specs=[pl.BlockSpec((4, BN_F, D), lambda i: (0, i, 0)),
              pl.BlockSpec((4, 128), lambda i: (0, 0))],
    out_specs=[pl.BlockSpec((BN_F, D), lambda i: (i, 0)),
               pl.BlockSpec((BN_F, D), lambda i: (i, 0))],
    out_shape=[jax.ShapeDtypeStruct((N, D), jnp.float32),
               jax.ShapeDtypeStruct((N, D), jnp.float32)],
)


def kernel(feat_user, feat_item, edge_index_u1, edge_index_u2,
           edge_index_i1, edge_index_i2, attn_l, attn_r,
           sem_W1, sem_b1, sem_W2):
    wu = jnp.stack([attn_r[0], attn_r[1], attn_l[2], attn_l[3]], axis=1)
    wi = jnp.stack([attn_l[0], attn_l[1], attn_r[2], attn_r[3]], axis=1)
    pu, pi = _proj(feat_user, feat_item, wu, wi)
    el_all = jnp.stack([pi[:, 0], pi[:, 1], pu[:, 2], pu[:, 3]]).reshape(-1)
    er_all = jnp.stack([pu[:, 0], pu[:, 1], pi[:, 2], pi[:, 3]]).reshape(-1)
    src_all = jnp.stack([edge_index_u1[0], edge_index_u2[0],
                         edge_index_i1[0], edge_index_i2[0]]).reshape(-1)
    dst_all = jnp.stack([edge_index_u1[1], edge_index_u2[1],
                         edge_index_i1[1], edge_index_i2[1]]).reshape(-1)
    feat_cat = jnp.concatenate([feat_item, feat_user], axis=0)
    out_part, _ = _sc_edge(el_all, er_all, src_all, dst_all, feat_cat)
    h, w = _comb(out_part, sem_W1, sem_b1.reshape(1, HID),
                 sem_W2.reshape(1, HID))
    emb_u, emb_i = _fin(h, w)
    return (emb_u, emb_i)


# speedup vs baseline: 1.9857x; 1.0081x over previous
import functools

import jax
import jax.numpy as jnp
from jax import lax
from jax.experimental import pallas as pl
from jax.experimental.pallas import tpu as pltpu
from jax.experimental.pallas import tpu_sc as plsc

N = 10000
D = 128
E = 320000
HID = 128

NC = 2
NS = 16
L = 16

EPT = E // NS
CH = 800
NCH = EPT // CH
RCHUNK = 80
NSUB = CH // RCHUNK
NPAD = 10240
ROWS_T = NPAD // NS
ZROWS = 128


def _sc_edge_body(el_hbm, er_hbm, src_hbm, dst_hbm, feat_hbm,
                  out_hbm, ee_hbm,
                  a_v, b_v, isrc_c, idst_c, ee_c, a_c, csrc0, cdst0,
                  csrc1, cdst1, rows0, rows1, shared_s, shared_out,
                  gsem0, gsem1, ssem0, ssem1):
    cid = lax.axis_index("c")
    sid = lax.axis_index("s")

    for lp in range(2):
        p = cid * 2 + lp
        ebase = p * E + sid * EPT

        def _za(i, c):
            a_v[pl.ds(i * 16, 16)] = jnp.zeros((16,), jnp.float32)
            return c
        lax.fori_loop(0, N // 16, _za, 0)

        @pl.when(sid == 0)
        def _():
            pltpu.sync_copy(a_v, shared_s)

        def _zr(i, c):
            for j in range(8):
                rows0[i, pl.ds(j * 16, 16)] = jnp.zeros((16,), jnp.float32)
            return c
        lax.fori_loop(0, RCHUNK, _zr, 0)
        for k in range(ROWS_T // RCHUNK):
            pltpu.sync_copy(
                rows0,
                shared_out.at[pl.ds(sid * ROWS_T + k * RCHUNK, RCHUNK)])

        pltpu.sync_copy(el_hbm.at[pl.ds(p * N, N)], a_v)
        pltpu.sync_copy(er_hbm.at[pl.ds(p * N, N)], b_v)
        plsc.subcore_barrier()

        def _sch(ci, c):
            off = ebase + ci * CH
            pltpu.sync_copy(src_hbm.at[pl.ds(off, CH)], isrc_c)
            pltpu.sync_copy(dst_hbm.at[pl.ds(off, CH)], idst_c)

            def _sg(k, c2):
                s16 = isrc_c[pl.ds(k * 16, 16)]
                d16 = idst_c[pl.ds(k * 16, 16)]
                ev = (plsc.load_gather(a_v, [s16])
                      + plsc.load_gather(b_v, [d16]))
                ev = jnp.where(ev >= 0.0, ev, 0.01 * ev)
                ee_c[pl.ds(k * 16, 16)] = jnp.exp(ev)
                return c2
            lax.fori_loop(0, CH // 16, _sg, 0, unroll=4)
            pltpu.sync_copy(ee_c, ee_hbm.at[pl.ds(off, CH)])

            sbufs = [(cdst0, ssem0), (cdst1, ssem1)]
            sscp = [None, None]
            for si in range(CH // RCHUNK):
                cdx, ssx = sbufs[si % 2]
                if sscp[si % 2] is not None:
                    sscp[si % 2].wait()
                soff = si * RCHUNK
                for k in range(RCHUNK // 16):
                    cdx[pl.ds(k * 16, 16)] = idst_c[pl.ds(soff + k * 16, 16)]
                sscp[si % 2] = pltpu.async_copy(
                    ee_c.at[pl.ds(soff, RCHUNK)], shared_s.at[cdx],
                    ssx, add=True)
            sscp[0].wait()
            sscp[1].wait()
            return c
        lax.fori_loop(0, NCH, _sch, 0)

        plsc.subcore_barrier()
        pltpu.sync_copy(shared_s, a_v)

        def _rcp(i, c):
            sl = pl.ds(i * 16, 16)
            a_v[sl] = 1.0 / (a_v[sl] + 1e-9)
            return c
        lax.fori_loop(0, N // 16, _rcp, 0)

        fbase = cid * N

        def _rch(ci, c):
            off = ebase + ci * CH
            pltpu.sync_copy(src_hbm.at[pl.ds(off, CH)], isrc_c)
            pltpu.sync_copy(dst_hbm.at[pl.ds(off, CH)], idst_c)
            pltpu.sync_copy(ee_hbm.at[pl.ds(off, CH)], ee_c)

            def _ag(k, c2):
                sl = pl.ds(k * 16, 16)
                d16 = idst_c[sl]
                a_c[sl] = ee_c[sl] * plsc.load_gather(a_v, [d16])
                return c2
            lax.fori_loop(0, CH // 16, _ag, 0, unroll=4)

            def _mk(soff, cs, cd):
                for k in range(RCHUNK // 16):
                    sl = pl.ds(k * 16, 16)
                    cs[sl] = isrc_c[pl.ds(soff + k * 16, 16)] + fbase
                    cd[sl] = idst_c[pl.ds(soff + k * 16, 16)]

            def _scale(soff, rows):
                def _sc(i, c3):
                    asp = plsc.load_gather(
                        a_c, [jnp.full((16,), soff + i, jnp.int32)])
                    for j in range(8):
                        sl = pl.ds(j * 16, 16)
                        rows[i, sl] = rows[i, sl] * asp
                    return c3
                lax.fori_loop(0, RCHUNK, _sc, 0, unroll=4)

            bufs = [(csrc0, cdst0, rows0, gsem0, ssem0),
                    (csrc1, cdst1, rows1, gsem1, ssem1)]
            nsub = CH // RCHUNK
            scp = [None, None]
            _mk(0, csrc0, cdst0)
            cp = pltpu.async_copy(feat_hbm.at[csrc0], rows0, gsem0)
            for si in range(nsub):
                cs, cd, rows, _, ssem = bufs[si % 2]
                cp.wait()
                if si + 1 < nsub:
                    ns, nd, nrows, nsem, _ = bufs[(si + 1) % 2]
                    if scp[(si + 1) % 2] is not None:
                        scp[(si + 1) % 2].wait()
                    _mk((si + 1) * RCHUNK, ns, nd)
                    cp = pltpu.async_copy(feat_hbm.at[ns], nrows, nsem)
                _scale(si * RCHUNK, rows)
                scp[si % 2] = pltpu.async_copy(
                    rows, shared_out.at[cd], ssem, add=True)
            scp[(nsub - 1) % 2].wait()
            scp[nsub % 2].wait()
            return c
        lax.fori_loop(0, NCH, _rch, 0)

        plsc.subcore_barrier()
        for k in range(ROWS_T // ZROWS):
            r0 = sid * ROWS_T + k * ZROWS
            pltpu.sync_copy(shared_out.at[pl.ds(r0, ZROWS)],
                            out_hbm.at[p, pl.ds(r0, ZROWS)])
        plsc.subcore_barrier()


_sc_edge = functools.partial(
    pl.kernel,
    out_type=(jax.ShapeDtypeStruct((4, NPAD, D), jnp.float32),
              jax.ShapeDtypeStruct((4 * E,), jnp.float32)),
    mesh=plsc.VectorSubcoreMesh(core_axis_name="c", subcore_axis_name="s"),
    compiler_params=pltpu.CompilerParams(needs_layout_passes=False),
    scratch_types=[
        pltpu.VMEM((N,), jnp.float32),
        pltpu.VMEM((N,), jnp.float32),
        pltpu.VMEM((CH,), jnp.int32),
        pltpu.VMEM((CH,), jnp.int32),
        pltpu.VMEM((CH,), jnp.float32),
        pltpu.VMEM((CH,), jnp.float32),
        pltpu.VMEM((RCHUNK,), jnp.int32),
        pltpu.VMEM((RCHUNK,), jnp.int32),
        pltpu.VMEM((RCHUNK,), jnp.int32),
        pltpu.VMEM((RCHUNK,), jnp.int32),
        pltpu.VMEM((RCHUNK, D), jnp.float32),
        pltpu.VMEM((RCHUNK, D), jnp.float32),
        pltpu.VMEM_SHARED((N,), jnp.float32),
        pltpu.VMEM_SHARED((NPAD, D), jnp.float32),
        pltpu.SemaphoreType.DMA,
        pltpu.SemaphoreType.DMA,
        pltpu.SemaphoreType.DMA,
        pltpu.SemaphoreType.DMA,
    ],
)(_sc_edge_body)


BN_A = 2000


def _proj_body(fu_ref, fi_ref, wu_ref, wi_ref, pu_ref, pi_ref):
    pu_ref[...] = jnp.dot(fu_ref[...], wu_ref[...],
                          preferred_element_type=jnp.float32)
    pi_ref[...] = jnp.dot(fi_ref[...], wi_ref[...],
                          preferred_element_type=jnp.float32)


_proj = pl.pallas_call(
    _proj_body,
    grid=(N // BN_A,),
    in_specs=[pl.BlockSpec((BN_A, D), lambda i: (i, 0)),
              pl.BlockSpec((BN_A, D), lambda i: (i, 0)),
              pl.BlockSpec((D, 4), lambda i: (0, 0)),
              pl.BlockSpec((D, 4), lambda i: (0, 0))],
    out_specs=[pl.BlockSpec((BN_A, 4), lambda i: (i, 0)),
               pl.BlockSpec((BN_A, 4), lambda i: (i, 0))],
    out_shape=[jax.ShapeDtypeStruct((N, 4), jnp.float32),
               jax.ShapeDtypeStruct((N, 4), jnp.float32)],
)

BN_B = 1000


def _comb_body(part_ref, w1_ref, b1_ref, w2_ref, h_ref, w_ref):
    i = pl.program_id(0)

    @pl.when(i == 0)
    def _():
        w_ref[...] = jnp.zeros_like(w_ref)

    for p in range(4):
        x = part_ref[p]
        h = jnp.where(x > 0.0, x, jnp.exp(x) - 1.0)
        h_ref[p] = h
        t = jnp.tanh(jnp.dot(h, w1_ref[...],
                             preferred_element_type=jnp.float32)
                     + b1_ref[...])
        contrib = jnp.sum(t * w2_ref[...])
        w_ref[pl.ds(p, 1), :] = w_ref[pl.ds(p, 1), :] + contrib


_comb = pl.pallas_call(
    _comb_body,
    grid=(N // BN_B,),
    in_specs=[pl.BlockSpec((4, BN_B, D), lambda i: (0, i, 0)),
              pl.BlockSpec((D, HID), lambda i: (0, 0)),
              pl.BlockSpec((1, HID), lambda i: (0, 0)),
              pl.BlockSpec((1, HID), lambda i: (0, 0))],
    out_specs=[pl.BlockSpec((4, BN_B, D), lambda i: (0, i, 0)),
               pl.BlockSpec((4, 128), lambda i: (0, 0))],
    out_shape=[jax.ShapeDtypeStruct((4, N, D), jnp.float32),
               jax.ShapeDtypeStruct((4, 128), jnp.float32)],
)

BN_F = 2000


def _fin_body(h_ref, w_ref, u_ref, i_ref):
    wv = w_ref[...] * (1.0 / N)
    wu0, wu1 = wv[0:1], wv[1:2]
    mu = jnp.maximum(wu0, wu1)
    e0, e1 = jnp.exp(wu0 - mu), jnp.exp(wu1 - mu)
    u_ref[...] = h_ref[0] * (e0 / (e0 + e1)) + h_ref[1] * (e1 / (e0 + e1))
    wi0, wi1 = wv[2:3], wv[3:4]
    mi = jnp.maximum(wi0, wi1)
    f0, f1 = jnp.exp(wi0 - mi), jnp.exp(wi1 - mi)
    i_ref[...] = h_ref[2] * (f0 / (f0 + f1)) + h_ref[3] * (f1 / (f0 + f1))


_fin = pl.pallas_call(
    _fin_body,
    grid=(N // BN_F,),
    in_specs=[pl.BlockSpec((4, BN_F, D), lambda i: (0, i, 0)),
              pl.BlockSpec((4, 128), lambda i: (0, 0))],
    out_specs=[pl.BlockSpec((BN_F, D), lambda i: (i, 0)),
               pl.BlockSpec((BN_F, D), lambda i: (i, 0))],
    out_shape=[jax.ShapeDtypeStruct((N, D), jnp.float32),
               jax.ShapeDtypeStruct((N, D), jnp.float32)],
)


def kernel(feat_user, feat_item, edge_index_u1, edge_index_u2,
           edge_index_i1, edge_index_i2, attn_l, attn_r,
           sem_W1, sem_b1, sem_W2):
    wu = jnp.stack([attn_r[0], attn_r[1], attn_l[2], attn_l[3]], axis=1)
    wi = jnp.stack([attn_l[0], attn_l[1], attn_r[2], attn_r[3]], axis=1)
    pu, pi = _proj(feat_user, feat_item, wu, wi)
    el_all = jnp.stack([pi[:, 0], pi[:, 1], pu[:, 2], pu[:, 3]]).reshape(-1)
    er_all = jnp.stack([pu[:, 0], pu[:, 1], pi[:, 2], pi[:, 3]]).reshape(-1)
    src_all = jnp.stack([edge_index_u1[0], edge_index_u2[0],
                         edge_index_i1[0], edge_index_i2[0]]).reshape(-1)
    dst_all = jnp.stack([edge_index_u1[1], edge_index_u2[1],
                         edge_index_i1[1], edge_index_i2[1]]).reshape(-1)
    feat_cat = jnp.concatenate([feat_item, feat_user], axis=0)
    out_part, _ = _sc_edge(el_all, er_all, src_all, dst_all, feat_cat)
    h, w = _comb(out_part, sem_W1, sem_b1.reshape(1, HID),
                 sem_W2.reshape(1, HID))
    emb_u, emb_i = _fin(h, w)
    return (emb_u, emb_i)
